# baseline (device time: 60587 ns/iter reference)
import jax
import jax.numpy as jnp
from jax import lax
from jax.experimental import pallas as pl
from jax.experimental.pallas import tpu as pltpu

B, S, H, Dh, Dr = 2, 256, 16, 64, 32
D = 1024
SCALE = (Dh + Dr) ** -0.5


def kernel(x, Wdkv, Wuk, Wuv, Wq, Wqr, Wkr, Wo):

    def body(x_ref, wdkv_ref, wuk_ref, wuv_ref, wq_ref, wqr_ref, wkr_ref,
             wo_ref, out_ref, kv_loc, kv_rem, o_ref, send_sem, recv_sem):
        my_x = lax.axis_index("x")
        my_y = lax.axis_index("y")
        my_z = lax.axis_index("z")
        peer = (1 - my_x, my_y, my_z)

        barrier = pltpu.get_barrier_semaphore()
        pl.semaphore_signal(barrier, inc=1, device_id=peer,
                            device_id_type=pl.DeviceIdType.MESH)
        pl.semaphore_wait(barrier, 1)

        wdkv = wdkv_ref[...].astype(jnp.bfloat16)
        wuk = wuk_ref[...].astype(jnp.bfloat16)
        wuv = wuv_ref[...].astype(jnp.bfloat16)

        for b in range(B):
            xb = x_ref[b].astype(jnp.bfloat16)
            cb = jnp.dot(xb, wdkv,
                         preferred_element_type=jnp.float32).astype(jnp.bfloat16)
            kv_loc[0, b] = jnp.dot(cb, wuk,
                                   preferred_element_type=jnp.float32
                                   ).astype(jnp.bfloat16)
            kv_loc[1, b] = jnp.dot(cb, wuv,
                                   preferred_element_type=jnp.float32
                                   ).astype(jnp.bfloat16)

        rdma = pltpu.make_async_remote_copy(
            src_ref=kv_loc, dst_ref=kv_rem,
            send_sem=send_sem, recv_sem=recv_sem,
            device_id=peer, device_id_type=pl.DeviceIdType.MESH)
        rdma.start()
        rdma.wait()

        wq = wq_ref[...].astype(jnp.bfloat16)
        wqr = wqr_ref[...].astype(jnp.bfloat16)
        wkr = wkr_ref[...].astype(jnp.bfloat16)
        wo = wo_ref[...].astype(jnp.bfloat16)

        for b in range(B):
            xb = x_ref[b].astype(jnp.bfloat16)
            Kb = kv_loc[0, b] + kv_rem[0, b]
            Vb = kv_loc[1, b] + kv_rem[1, b]
            Qb = jnp.dot(xb, wq,
                         preferred_element_type=jnp.float32).astype(jnp.bfloat16)
            Qrb = jnp.dot(xb, wqr,
                          preferred_element_type=jnp.float32).astype(jnp.bfloat16)
            Krb = jnp.dot(xb, wkr,
                          preferred_element_type=jnp.float32).astype(jnp.bfloat16)
            for h in range(H):
                q = Qb[:, h * Dh:(h + 1) * Dh]
                k = Kb[:, h * Dh:(h + 1) * Dh]
                v = Vb[:, h * Dh:(h + 1) * Dh]
                qr = Qrb[:, h * Dr:(h + 1) * Dr]
                s = (lax.dot_general(q, k, (((1,), (1,)), ((), ())),
                                     preferred_element_type=jnp.float32)
                     + lax.dot_general(qr, Krb, (((1,), (1,)), ((), ())),
                                       preferred_element_type=jnp.float32)
                     ) * SCALE
                m = jnp.max(s, axis=-1, keepdims=True)
                p = jnp.exp(s - m)
                p = p / jnp.sum(p, axis=-1, keepdims=True)
                o_ref[b, :, h * Dh:(h + 1) * Dh] = jnp.dot(
                    p.astype(jnp.bfloat16), v,
                    preferred_element_type=jnp.float32).astype(jnp.bfloat16)
            out_ref[b] = jnp.dot(o_ref[b], wo,
                                 preferred_element_type=jnp.float32)

    return pl.pallas_call(
        body,
        out_shape=jax.ShapeDtypeStruct((B, S, D), jnp.float32),
        in_specs=[pl.BlockSpec(memory_space=pltpu.VMEM)] * 8,
        out_specs=pl.BlockSpec(memory_space=pltpu.VMEM),
        scratch_shapes=[
            pltpu.VMEM((2, B, S, H * Dh), jnp.bfloat16),
            pltpu.VMEM((2, B, S, H * Dh), jnp.bfloat16),
            pltpu.VMEM((B, S, H * Dh), jnp.bfloat16),
            pltpu.SemaphoreType.DMA,
            pltpu.SemaphoreType.DMA,
        ],
        compiler_params=pltpu.CompilerParams(collective_id=0),
    )(x, Wdkv, Wuk, Wuv, Wq, Wqr, Wkr, Wo)


# device time: 40395 ns/iter; 1.4999x vs baseline; 1.4999x over previous
import jax
import jax.numpy as jnp
from jax import lax
from jax.experimental import pallas as pl
from jax.experimental.pallas import tpu as pltpu

B, S, H, Dh, Dr = 2, 256, 16, 64, 32
D = 1024
DC = 64
SCALE = (Dh + Dr) ** -0.5


def kernel(x, Wdkv, Wuk, Wuv, Wq, Wqr, Wkr, Wo):

    def body(x_ref, wdkv_ref, wuk_ref, wuv_ref, wq_ref, wqr_ref, wkr_ref,
             wo_ref, out_ref, c_loc, c_rem, kvw_loc, kvw_rem, o_ref,
             send_sems, recv_sems):
        my_x = lax.axis_index("x")
        my_y = lax.axis_index("y")
        my_z = lax.axis_index("z")
        peer = (1 - my_x, my_y, my_z)

        barrier = pltpu.get_barrier_semaphore()
        pl.semaphore_signal(barrier, inc=1, device_id=peer,
                            device_id_type=pl.DeviceIdType.MESH)
        pl.semaphore_wait(barrier, 1)

        wdkv = wdkv_ref[...].astype(jnp.bfloat16)
        kvw_loc[0] = wuk_ref[...].astype(jnp.bfloat16)
        kvw_loc[1] = wuv_ref[...].astype(jnp.bfloat16)
        for b in range(B):
            c_loc[b] = jnp.dot(x_ref[b].astype(jnp.bfloat16), wdkv,
                               preferred_element_type=jnp.float32
                               ).astype(jnp.bfloat16)

        rdma_c = pltpu.make_async_remote_copy(
            src_ref=c_loc, dst_ref=c_rem,
            send_sem=send_sems.at[0], recv_sem=recv_sems.at[0],
            device_id=peer, device_id_type=pl.DeviceIdType.MESH)
        rdma_w = pltpu.make_async_remote_copy(
            src_ref=kvw_loc, dst_ref=kvw_rem,
            send_sem=send_sems.at[1], recv_sem=recv_sems.at[1],
            device_id=peer, device_id_type=pl.DeviceIdType.MESH)
        rdma_c.start()
        rdma_w.start()

        wq = wq_ref[...].astype(jnp.bfloat16)
        wqr = wqr_ref[...].astype(jnp.bfloat16)
        wkr = wkr_ref[...].astype(jnp.bfloat16)
        wuk = kvw_loc[0]
        wuv = kvw_loc[1]
        Q = []
        Qr = []
        Kr = []
        Kp = []
        Vp = []
        for b in range(B):
            xb = x_ref[b].astype(jnp.bfloat16)
            Q.append(jnp.dot(xb, wq,
                             preferred_element_type=jnp.float32
                             ).astype(jnp.bfloat16))
            Qr.append(jnp.dot(xb, wqr,
                              preferred_element_type=jnp.float32
                              ).astype(jnp.bfloat16))
            Kr.append(jnp.dot(xb, wkr,
                              preferred_element_type=jnp.float32
                              ).astype(jnp.bfloat16))
            Kp.append(jnp.dot(c_loc[b], wuk, preferred_element_type=jnp.float32))
            Vp.append(jnp.dot(c_loc[b], wuv, preferred_element_type=jnp.float32))

        rdma_c.wait()
        rdma_w.wait()

        wo = wo_ref[...].astype(jnp.bfloat16)
        for b in range(B):
            Kb = (Kp[b] + jnp.dot(c_rem[b], kvw_rem[0],
                                  preferred_element_type=jnp.float32)
                  ).astype(jnp.bfloat16)
            Vb = (Vp[b] + jnp.dot(c_rem[b], kvw_rem[1],
                                  preferred_element_type=jnp.float32)
                  ).astype(jnp.bfloat16)
            for h in range(H):
                q = Q[b][:, h * Dh:(h + 1) * Dh]
                k = Kb[:, h * Dh:(h + 1) * Dh]
                v = Vb[:, h * Dh:(h + 1) * Dh]
                qr = Qr[b][:, h * Dr:(h + 1) * Dr]
                s = (lax.dot_general(q, k, (((1,), (1,)), ((), ())),
                                     preferred_element_type=jnp.float32)
                     + lax.dot_general(qr, Kr[b], (((1,), (1,)), ((), ())),
                                       preferred_element_type=jnp.float32)
                     ) * SCALE
                m = jnp.max(s, axis=-1, keepdims=True)
                p = jnp.exp(s - m)
                p = p / jnp.sum(p, axis=-1, keepdims=True)
                o_ref[b, :, h * Dh:(h + 1) * Dh] = jnp.dot(
                    p.astype(jnp.bfloat16), v,
                    preferred_element_type=jnp.float32).astype(jnp.bfloat16)
            out_ref[b] = jnp.dot(o_ref[b], wo,
                                 preferred_element_type=jnp.float32)

    return pl.pallas_call(
        body,
        out_shape=jax.ShapeDtypeStruct((B, S, D), jnp.float32),
        in_specs=[pl.BlockSpec(memory_space=pltpu.VMEM)] * 8,
        out_specs=pl.BlockSpec(memory_space=pltpu.VMEM),
        scratch_shapes=[
            pltpu.VMEM((B, S, DC), jnp.bfloat16),
            pltpu.VMEM((B, S, DC), jnp.bfloat16),
            pltpu.VMEM((2, DC, D), jnp.bfloat16),
            pltpu.VMEM((2, DC, D), jnp.bfloat16),
            pltpu.VMEM((B, S, H * Dh), jnp.bfloat16),
            pltpu.SemaphoreType.DMA((2,)),
            pltpu.SemaphoreType.DMA((2,)),
        ],
        compiler_params=pltpu.CompilerParams(collective_id=0),
    )(x, Wdkv, Wuk, Wuv, Wq, Wqr, Wkr, Wo)


# device time: 34756 ns/iter; 1.7432x vs baseline; 1.1622x over previous
import jax
import jax.numpy as jnp
from jax import lax
from jax.experimental import pallas as pl
from jax.experimental.pallas import tpu as pltpu

B, S, H, Dh, Dr = 2, 256, 16, 64, 32
D = 1024
DC = 64
G = 4
HG = H // G
GC = HG * Dh
GR = HG * Dr
SCALE = (Dh + Dr) ** -0.5


def kernel(x, Wdkv, Wuk, Wuv, Wq, Wqr, Wkr, Wo):

    def body(x_ref, wdkv_ref, wuk_ref, wuv_ref, wq_ref, wqr_ref, wkr_ref,
             wo_ref, out_ref, c_loc, c_rem, kvw_loc, kvw_rem, o_parts,
             x_send_sems, x_recv_sems, o_send_sems, o_recv_sems):
        my_x = lax.axis_index("x")
        my_y = lax.axis_index("y")
        my_z = lax.axis_index("z")
        my_g = 2 * my_y + my_z
        peer_x = (1 - my_x, my_y, my_z)
        o_peers = [
            (my_x, 1 - my_y, my_z),
            (my_x, my_y, 1 - my_z),
            (my_x, 1 - my_y, 1 - my_z),
        ]
        slot_g = [
            2 * (1 - my_y) + my_z,
            2 * my_y + (1 - my_z),
            2 * (1 - my_y) + (1 - my_z),
            my_g,
        ]

        barrier = pltpu.get_barrier_semaphore()
        for nbr in [peer_x] + o_peers:
            pl.semaphore_signal(barrier, inc=1, device_id=nbr,
                                device_id_type=pl.DeviceIdType.MESH)
        pl.semaphore_wait(barrier, 4)

        wdkv = wdkv_ref[...].astype(jnp.bfloat16)
        kvw_loc[0] = wuk_ref[:, pl.ds(my_g * GC, GC)].astype(jnp.bfloat16)
        kvw_loc[1] = wuv_ref[:, pl.ds(my_g * GC, GC)].astype(jnp.bfloat16)
        for b in range(B):
            c_loc[b] = jnp.dot(x_ref[b].astype(jnp.bfloat16), wdkv,
                               preferred_element_type=jnp.float32
                               ).astype(jnp.bfloat16)

        rdma_c = pltpu.make_async_remote_copy(
            src_ref=c_loc, dst_ref=c_rem,
            send_sem=x_send_sems.at[0], recv_sem=x_recv_sems.at[0],
            device_id=peer_x, device_id_type=pl.DeviceIdType.MESH)
        rdma_w = pltpu.make_async_remote_copy(
            src_ref=kvw_loc, dst_ref=kvw_rem,
            send_sem=x_send_sems.at[1], recv_sem=x_recv_sems.at[1],
            device_id=peer_x, device_id_type=pl.DeviceIdType.MESH)
        rdma_c.start()
        rdma_w.start()

        wq_g = wq_ref[:, pl.ds(my_g * GC, GC)].astype(jnp.bfloat16)
        wqr_g = wqr_ref[:, pl.ds(my_g * GR, GR)].astype(jnp.bfloat16)
        wkr = wkr_ref[...].astype(jnp.bfloat16)
        Q = []
        Qr = []
        Kr = []
        Kp = []
        Vp = []
        for b in range(B):
            xb = x_ref[b].astype(jnp.bfloat16)
            Q.append(jnp.dot(xb, wq_g,
                             preferred_element_type=jnp.float32
                             ).astype(jnp.bfloat16))
            Qr.append(jnp.dot(xb, wqr_g,
                              preferred_element_type=jnp.float32
                              ).astype(jnp.bfloat16))
            Kr.append(jnp.dot(xb, wkr,
                              preferred_element_type=jnp.float32
                              ).astype(jnp.bfloat16))
            Kp.append(jnp.dot(c_loc[b], kvw_loc[0],
                              preferred_element_type=jnp.float32))
            Vp.append(jnp.dot(c_loc[b], kvw_loc[1],
                              preferred_element_type=jnp.float32))

        rdma_c.wait()
        rdma_w.wait()

        for b in range(B):
            Kb = (Kp[b] + jnp.dot(c_rem[b], kvw_rem[0],
                                  preferred_element_type=jnp.float32)
                  ).astype(jnp.bfloat16)
            Vb = (Vp[b] + jnp.dot(c_rem[b], kvw_rem[1],
                                  preferred_element_type=jnp.float32)
                  ).astype(jnp.bfloat16)
            for h in range(HG):
                q = Q[b][:, h * Dh:(h + 1) * Dh]
                k = Kb[:, h * Dh:(h + 1) * Dh]
                v = Vb[:, h * Dh:(h + 1) * Dh]
                qr = Qr[b][:, h * Dr:(h + 1) * Dr]
                s = (lax.dot_general(q, k, (((1,), (1,)), ((), ())),
                                     preferred_element_type=jnp.float32)
                     + lax.dot_general(qr, Kr[b], (((1,), (1,)), ((), ())),
                                       preferred_element_type=jnp.float32)
                     ) * SCALE
                m = jnp.max(s, axis=-1, keepdims=True)
                p = jnp.exp(s - m)
                p = p / jnp.sum(p, axis=-1, keepdims=True)
                o_parts[3, b, :, h * Dh:(h + 1) * Dh] = jnp.dot(
                    p.astype(jnp.bfloat16), v,
                    preferred_element_type=jnp.float32).astype(jnp.bfloat16)

        o_rdmas = []
        for j, nbr in enumerate(o_peers):
            r = pltpu.make_async_remote_copy(
                src_ref=o_parts.at[3], dst_ref=o_parts.at[j],
                send_sem=o_send_sems.at[j], recv_sem=o_recv_sems.at[j],
                device_id=nbr, device_id_type=pl.DeviceIdType.MESH)
            r.start()
            o_rdmas.append(r)

        for b in range(B):
            out_ref[b] = jnp.dot(
                o_parts[3, b],
                wo_ref[pl.ds(my_g * GC, GC), :].astype(jnp.bfloat16),
                preferred_element_type=jnp.float32)

        for r in o_rdmas:
            r.wait()

        for j in range(3):
            wo_j = wo_ref[pl.ds(slot_g[j] * GC, GC), :].astype(jnp.bfloat16)
            for b in range(B):
                out_ref[b] += jnp.dot(o_parts[j, b], wo_j,
                                      preferred_element_type=jnp.float32)

    return pl.pallas_call(
        body,
        out_shape=jax.ShapeDtypeStruct((B, S, D), jnp.float32),
        in_specs=[pl.BlockSpec(memory_space=pltpu.VMEM)] * 8,
        out_specs=pl.BlockSpec(memory_space=pltpu.VMEM),
        scratch_shapes=[
            pltpu.VMEM((B, S, DC), jnp.bfloat16),
            pltpu.VMEM((B, S, DC), jnp.bfloat16),
            pltpu.VMEM((2, DC, GC), jnp.bfloat16),
            pltpu.VMEM((2, DC, GC), jnp.bfloat16),
            pltpu.VMEM((4, B, S, GC), jnp.bfloat16),
            pltpu.SemaphoreType.DMA((2,)),
            pltpu.SemaphoreType.DMA((2,)),
            pltpu.SemaphoreType.DMA((3,)),
            pltpu.SemaphoreType.DMA((3,)),
        ],
        compiler_params=pltpu.CompilerParams(collective_id=0),
    )(x, Wdkv, Wuk, Wuv, Wq, Wqr, Wkr, Wo)
